# baseline (speedup 1.0000x reference)
import dataclasses
import functools

import jax
import jax.numpy as jnp
from jax import lax
from jax.experimental import pallas as pl
from jax.experimental.pallas import tpu as pltpu
from jax.experimental.pallas import tpu_sc as plsc

B = 4096
INPUT_DIM = 2000
HIDDEN = 1024
LATENT = 256
K = 8192
CC = 0.25

SC_CORES = 2
SC_SUBCORES = 16
SC_LANES = 16
SC_TILES = SC_CORES * SC_SUBCORES
ROWS_PER_TILE = B // SC_TILES

BM = 512
KCHUNK = 2048


def _encoder_body(x_ref, w1_ref, b1_ref, w2_ref, b2_ref, w3_ref, b3_ref,
                  lat_ref):
    h = jnp.maximum(jnp.dot(x_ref[...], w1_ref[...]) + b1_ref[...], 0.0)
    h = jnp.maximum(jnp.dot(h, w2_ref[...]) + b2_ref[...], 0.0)
    lat_ref[...] = jnp.tanh(jnp.dot(h, w3_ref[...]) + b3_ref[...])


def _encoder(x, We1, be1, We2, be2, We3, be3):
    grid = (B // BM,)
    return pl.pallas_call(
        _encoder_body,
        grid=grid,
        in_specs=[
            pl.BlockSpec((BM, INPUT_DIM), lambda i: (i, 0)),
            pl.BlockSpec((INPUT_DIM, HIDDEN), lambda i: (0, 0)),
            pl.BlockSpec((1, HIDDEN), lambda i: (0, 0)),
            pl.BlockSpec((HIDDEN, HIDDEN), lambda i: (0, 0)),
            pl.BlockSpec((1, HIDDEN), lambda i: (0, 0)),
            pl.BlockSpec((HIDDEN, LATENT), lambda i: (0, 0)),
            pl.BlockSpec((1, LATENT), lambda i: (0, 0)),
        ],
        out_specs=pl.BlockSpec((BM, LATENT), lambda i: (i, 0)),
        out_shape=jax.ShapeDtypeStruct((B, LATENT), jnp.float32),
    )(x, We1, be1[None, :], We2, be2[None, :], We3, be3[None, :])


def _vq_body(lat_ref, cb_ref, idx_ref):
    l = lat_ref[...]
    a = jnp.sum(l * l, axis=1, keepdims=True)

    def chunk(c, carry):
        minval, minidx = carry
        cb = cb_ref[pl.ds(c * KCHUNK, KCHUNK), :]
        bsq = jnp.sum(cb * cb, axis=1)[None, :]
        m = lax.dot_general(l, cb, (((1,), (1,)), ((), ())))
        d = (a + bsq) - 2.0 * m
        cmin = jnp.min(d, axis=1, keepdims=True)
        iota = lax.broadcasted_iota(jnp.int32, d.shape, 1)
        cidx = jnp.min(jnp.where(d == cmin, iota, KCHUNK), axis=1,
                       keepdims=True) + c * KCHUNK
        upd = cmin < minval
        return (jnp.where(upd, cmin, minval), jnp.where(upd, cidx, minidx))

    init = (jnp.full((BM, 1), jnp.inf, jnp.float32),
            jnp.zeros((BM, 1), jnp.int32))
    _, minidx = lax.fori_loop(0, K // KCHUNK, chunk, init)
    idx_ref[...] = minidx


def _vq_argmin(latent, codebook):
    grid = (B // BM,)
    return pl.pallas_call(
        _vq_body,
        grid=grid,
        in_specs=[
            pl.BlockSpec((BM, LATENT), lambda i: (i, 0)),
            pl.BlockSpec((K, LATENT), lambda i: (0, 0)),
        ],
        out_specs=pl.BlockSpec((BM, 1), lambda i: (i, 0)),
        out_shape=jax.ShapeDtypeStruct((B, 1), jnp.int32),
    )(latent, codebook)


def _sc_gather_hist(codebook, indices, zeros_k):
    mesh = plsc.VectorSubcoreMesh(core_axis_name="c", subcore_axis_name="s")
    cp = pltpu.CompilerParams()
    if "needs_layout_passes" in pltpu.CompilerParams.__dataclass_fields__:
        cp = dataclasses.replace(cp, needs_layout_passes=False)

    @functools.partial(
        pl.kernel,
        mesh=mesh,
        compiler_params=cp,
        out_type=[
            jax.ShapeDtypeStruct((B, LATENT), jnp.float32),
            jax.ShapeDtypeStruct((SC_TILES, K), jnp.float32),
        ],
        scratch_types=[
            pltpu.VMEM((ROWS_PER_TILE,), jnp.int32),
            pltpu.VMEM((ROWS_PER_TILE, LATENT), jnp.float32),
            pltpu.VMEM((K,), jnp.float32),
            pltpu.SemaphoreType.DMA,
        ],
    )
    def k(cb_hbm, idx_hbm, zero_hbm, q_hbm, hist_hbm, idx_v, rows_v,
          cnt_v, sem):
        wid = lax.axis_index("s") * SC_CORES + lax.axis_index("c")
        base = wid * ROWS_PER_TILE
        pltpu.sync_copy(idx_hbm.at[pl.ds(base, ROWS_PER_TILE)], idx_v)
        gather = pltpu.async_copy(cb_hbm.at[idx_v], rows_v, sem)
        pltpu.sync_copy(zero_hbm, cnt_v)
        ones = jnp.full((SC_LANES,), 1.0, jnp.float32)
        for j in range(ROWS_PER_TILE // SC_LANES):
            iv = idx_v[pl.ds(j * SC_LANES, SC_LANES)]
            plsc.addupdate_scatter(cnt_v, [iv], ones)
        pltpu.sync_copy(cnt_v, hist_hbm.at[wid])
        gather.wait()
        pltpu.sync_copy(rows_v, q_hbm.at[pl.ds(base, ROWS_PER_TILE)])

    return k(codebook, indices, zeros_k)


def _decoder_body(q_ref, lat_ref, x_ref, w1_ref, b1_ref, w2_ref, b2_ref,
                  w3_ref, b3_ref, rec_ref, sse_ref):
    q = q_ref[...]
    l = lat_ref[...]
    qst = l + (q - l)
    h = jnp.maximum(jnp.dot(qst, w1_ref[...]) + b1_ref[...], 0.0)
    h = jnp.maximum(jnp.dot(h, w2_ref[...]) + b2_ref[...], 0.0)
    rec = jnp.dot(h, w3_ref[...]) + b3_ref[...]
    rec_ref[...] = rec

    dvq = q - l
    sse_vq = jnp.sum(dvq * dvq)
    drec = rec - x_ref[...]
    sse_rec = jnp.sum(drec * drec)

    @pl.when(pl.program_id(0) == 0)
    def _():
        sse_ref[...] = jnp.zeros((8, 128), jnp.float32)

    row = lax.broadcasted_iota(jnp.int32, (8, 128), 0)
    col = lax.broadcasted_iota(jnp.int32, (8, 128), 1)
    upd = jnp.where((row == 0) & (col == 0), sse_vq, 0.0)
    upd = upd + jnp.where((row == 0) & (col == 1), sse_rec, 0.0)
    sse_ref[...] = sse_ref[...] + upd


def _decoder(quantized, latent, x, Wd1, bd1, Wd2, bd2, Wd3, bd3):
    grid = (B // BM,)
    return pl.pallas_call(
        _decoder_body,
        grid=grid,
        in_specs=[
            pl.BlockSpec((BM, LATENT), lambda i: (i, 0)),
            pl.BlockSpec((BM, LATENT), lambda i: (i, 0)),
            pl.BlockSpec((BM, INPUT_DIM), lambda i: (i, 0)),
            pl.BlockSpec((LATENT, HIDDEN), lambda i: (0, 0)),
            pl.BlockSpec((1, HIDDEN), lambda i: (0, 0)),
            pl.BlockSpec((HIDDEN, HIDDEN), lambda i: (0, 0)),
            pl.BlockSpec((1, HIDDEN), lambda i: (0, 0)),
            pl.BlockSpec((HIDDEN, INPUT_DIM), lambda i: (0, 0)),
            pl.BlockSpec((1, INPUT_DIM), lambda i: (0, 0)),
        ],
        out_specs=[
            pl.BlockSpec((BM, INPUT_DIM), lambda i: (i, 0)),
            pl.BlockSpec((8, 128), lambda i: (0, 0)),
        ],
        out_shape=[
            jax.ShapeDtypeStruct((B, INPUT_DIM), jnp.float32),
            jax.ShapeDtypeStruct((8, 128), jnp.float32),
        ],
    )(quantized, latent, x, Wd1, bd1[None, :], Wd2, bd2[None, :], Wd3,
      bd3[None, :])


def _stats_body(hist_ref, out_ref):
    counts = jnp.sum(hist_ref[...], axis=0, keepdims=True)
    avg = counts * (1.0 / B)
    ent = avg * jnp.log(avg + 1e-10)
    perp = jnp.exp(-jnp.sum(ent))
    out_ref[...] = jnp.full((8, 128), perp, jnp.float32)


def _stats(hist):
    return pl.pallas_call(
        _stats_body,
        in_specs=[pl.BlockSpec((SC_TILES, K), lambda: (0, 0))],
        out_specs=pl.BlockSpec((8, 128), lambda: (0, 0)),
        out_shape=jax.ShapeDtypeStruct((8, 128), jnp.float32),
        grid=(),
    )(hist)


def kernel(x, We1, be1, We2, be2, We3, be3, codebook,
           Wd1, bd1, Wd2, bd2, Wd3, bd3):
    latent = _encoder(x, We1, be1, We2, be2, We3, be3)
    idx2d = _vq_argmin(latent, codebook)
    indices = idx2d.reshape(B)
    zeros_k = jnp.zeros((K,), jnp.float32)
    quantized, hist = _sc_gather_hist(codebook, indices, zeros_k)
    reconstruction, sse = _decoder(quantized, latent, x,
                                   Wd1, bd1, Wd2, bd2, Wd3, bd3)
    perp = _stats(hist)[0, 0]

    mean_vq = sse[0, 0] / (B * LATENT)
    vq_loss = mean_vq + CC * mean_vq
    recon_loss = sse[0, 1] / (B * INPUT_DIM)
    total_loss = vq_loss + recon_loss
    return (reconstruction, indices, vq_loss, recon_loss, total_loss, perp)


# speedup vs baseline: 1.0055x; 1.0055x over previous
import dataclasses
import functools

import jax
import jax.numpy as jnp
from jax import lax
from jax.experimental import pallas as pl
from jax.experimental.pallas import tpu as pltpu
from jax.experimental.pallas import tpu_sc as plsc

B = 4096
INPUT_DIM = 2000
HIDDEN = 1024
LATENT = 256
K = 8192
CC = 0.25

SC_CORES = 2
SC_SUBCORES = 16
SC_LANES = 16
SC_TILES = SC_CORES * SC_SUBCORES
ROWS_PER_TILE = B // SC_TILES

BM = 512
KCHUNK = 2048


def _encoder_body(x_ref, w1_ref, b1_ref, w2_ref, b2_ref, w3_ref, b3_ref,
                  lat_ref):
    h = jnp.maximum(jnp.dot(x_ref[...], w1_ref[...]) + b1_ref[...], 0.0)
    h = jnp.maximum(jnp.dot(h, w2_ref[...]) + b2_ref[...], 0.0)
    lat_ref[...] = jnp.tanh(jnp.dot(h, w3_ref[...]) + b3_ref[...])


def _encoder(x, We1, be1, We2, be2, We3, be3):
    grid = (B // BM,)
    return pl.pallas_call(
        _encoder_body,
        grid=grid,
        in_specs=[
            pl.BlockSpec((BM, INPUT_DIM), lambda i: (i, 0)),
            pl.BlockSpec((INPUT_DIM, HIDDEN), lambda i: (0, 0)),
            pl.BlockSpec((1, HIDDEN), lambda i: (0, 0)),
            pl.BlockSpec((HIDDEN, HIDDEN), lambda i: (0, 0)),
            pl.BlockSpec((1, HIDDEN), lambda i: (0, 0)),
            pl.BlockSpec((HIDDEN, LATENT), lambda i: (0, 0)),
            pl.BlockSpec((1, LATENT), lambda i: (0, 0)),
        ],
        out_specs=pl.BlockSpec((BM, LATENT), lambda i: (i, 0)),
        out_shape=jax.ShapeDtypeStruct((B, LATENT), jnp.float32),
    )(x, We1, be1[None, :], We2, be2[None, :], We3, be3[None, :])


def _vq_body(lat_ref, cb_ref, idx_ref):
    l = lat_ref[...]
    a = jnp.sum(l * l, axis=1, keepdims=True)

    def chunk(c, carry):
        minval, minidx = carry
        cb = cb_ref[pl.ds(c * KCHUNK, KCHUNK), :]
        bsq = jnp.sum(cb * cb, axis=1)[None, :]
        m = lax.dot_general(l, cb, (((1,), (1,)), ((), ())))
        d = (a + bsq) - 2.0 * m
        cmin = jnp.min(d, axis=1, keepdims=True)
        iota = lax.broadcasted_iota(jnp.int32, d.shape, 1)
        cidx = jnp.min(jnp.where(d == cmin, iota, KCHUNK), axis=1,
                       keepdims=True) + c * KCHUNK
        upd = cmin < minval
        return (jnp.where(upd, cmin, minval), jnp.where(upd, cidx, minidx))

    init = (jnp.full((BM, 1), jnp.inf, jnp.float32),
            jnp.zeros((BM, 1), jnp.int32))
    _, minidx = lax.fori_loop(0, K // KCHUNK, chunk, init)
    idx_ref[...] = minidx


def _vq_argmin(latent, codebook):
    grid = (B // BM,)
    return pl.pallas_call(
        _vq_body,
        grid=grid,
        in_specs=[
            pl.BlockSpec((BM, LATENT), lambda i: (i, 0)),
            pl.BlockSpec((K, LATENT), lambda i: (0, 0)),
        ],
        out_specs=pl.BlockSpec((BM, 1), lambda i: (i, 0)),
        out_shape=jax.ShapeDtypeStruct((B, 1), jnp.int32),
    )(latent, codebook)


def _sc_gather_hist(codebook, indices, zeros_k):
    mesh = plsc.VectorSubcoreMesh(core_axis_name="c", subcore_axis_name="s")
    cp = pltpu.CompilerParams()
    if "needs_layout_passes" in pltpu.CompilerParams.__dataclass_fields__:
        cp = dataclasses.replace(cp, needs_layout_passes=False)

    @functools.partial(
        pl.kernel,
        mesh=mesh,
        compiler_params=cp,
        out_type=[
            jax.ShapeDtypeStruct((B, LATENT), jnp.float32),
            jax.ShapeDtypeStruct((SC_TILES, K), jnp.float32),
        ],
        scratch_types=[
            pltpu.VMEM((ROWS_PER_TILE,), jnp.int32),
            pltpu.VMEM((ROWS_PER_TILE, LATENT), jnp.float32),
            pltpu.VMEM((K,), jnp.float32),
            pltpu.SemaphoreType.DMA,
        ],
    )
    def k(cb_hbm, idx_hbm, zero_hbm, q_hbm, hist_hbm, idx_v, rows_v,
          cnt_v, sem):
        wid = lax.axis_index("s") * SC_CORES + lax.axis_index("c")
        base = wid * ROWS_PER_TILE
        pltpu.sync_copy(idx_hbm.at[pl.ds(base, ROWS_PER_TILE)], idx_v)
        gather = pltpu.async_copy(cb_hbm.at[idx_v], rows_v, sem)
        pltpu.sync_copy(zero_hbm, cnt_v)
        ones = jnp.full((SC_LANES,), 1.0, jnp.float32)
        for j in range(ROWS_PER_TILE // SC_LANES):
            iv = idx_v[pl.ds(j * SC_LANES, SC_LANES)]
            plsc.addupdate_scatter(cnt_v, [iv], ones)
        pltpu.sync_copy(cnt_v, hist_hbm.at[wid])
        gather.wait()
        pltpu.sync_copy(rows_v, q_hbm.at[pl.ds(base, ROWS_PER_TILE)])

    return k(codebook, indices, zeros_k)


def _decoder_body(q_ref, lat_ref, x_ref, w1_ref, b1_ref, w2_ref, b2_ref,
                  w3_ref, b3_ref, rec_ref, sse_ref):
    q = q_ref[...]
    l = lat_ref[...]
    qst = l + (q - l)
    f32 = jnp.float32
    h = jnp.maximum(
        jnp.dot(qst.astype(jnp.bfloat16), w1_ref[...],
                preferred_element_type=f32) + b1_ref[...], 0.0)
    h = jnp.maximum(
        jnp.dot(h.astype(jnp.bfloat16), w2_ref[...],
                preferred_element_type=f32) + b2_ref[...], 0.0)
    rec = jnp.dot(h.astype(jnp.bfloat16), w3_ref[...],
                  preferred_element_type=f32) + b3_ref[...]
    rec_ref[...] = rec

    dvq = q - l
    sse_vq = jnp.sum(dvq * dvq)
    drec = rec - x_ref[...]
    sse_rec = jnp.sum(drec * drec)

    @pl.when(pl.program_id(0) == 0)
    def _():
        sse_ref[...] = jnp.zeros((8, 128), jnp.float32)

    row = lax.broadcasted_iota(jnp.int32, (8, 128), 0)
    col = lax.broadcasted_iota(jnp.int32, (8, 128), 1)
    upd = jnp.where((row == 0) & (col == 0), sse_vq, 0.0)
    upd = upd + jnp.where((row == 0) & (col == 1), sse_rec, 0.0)
    sse_ref[...] = sse_ref[...] + upd


def _decoder(quantized, latent, x, Wd1, bd1, Wd2, bd2, Wd3, bd3):
    grid = (B // BM,)
    return pl.pallas_call(
        _decoder_body,
        grid=grid,
        in_specs=[
            pl.BlockSpec((BM, LATENT), lambda i: (i, 0)),
            pl.BlockSpec((BM, LATENT), lambda i: (i, 0)),
            pl.BlockSpec((BM, INPUT_DIM), lambda i: (i, 0)),
            pl.BlockSpec((LATENT, HIDDEN), lambda i: (0, 0)),
            pl.BlockSpec((1, HIDDEN), lambda i: (0, 0)),
            pl.BlockSpec((HIDDEN, HIDDEN), lambda i: (0, 0)),
            pl.BlockSpec((1, HIDDEN), lambda i: (0, 0)),
            pl.BlockSpec((HIDDEN, INPUT_DIM), lambda i: (0, 0)),
            pl.BlockSpec((1, INPUT_DIM), lambda i: (0, 0)),
        ],
        out_specs=[
            pl.BlockSpec((BM, INPUT_DIM), lambda i: (i, 0)),
            pl.BlockSpec((8, 128), lambda i: (0, 0)),
        ],
        out_shape=[
            jax.ShapeDtypeStruct((B, INPUT_DIM), jnp.float32),
            jax.ShapeDtypeStruct((8, 128), jnp.float32),
        ],
    )(quantized, latent, x,
      Wd1.astype(jnp.bfloat16), bd1[None, :],
      Wd2.astype(jnp.bfloat16), bd2[None, :],
      Wd3.astype(jnp.bfloat16), bd3[None, :])


def _stats_body(hist_ref, out_ref):
    counts = jnp.sum(hist_ref[...], axis=0, keepdims=True)
    avg = counts * (1.0 / B)
    ent = avg * jnp.log(avg + 1e-10)
    perp = jnp.exp(-jnp.sum(ent))
    out_ref[...] = jnp.full((8, 128), perp, jnp.float32)


def _stats(hist):
    return pl.pallas_call(
        _stats_body,
        in_specs=[pl.BlockSpec((SC_TILES, K), lambda: (0, 0))],
        out_specs=pl.BlockSpec((8, 128), lambda: (0, 0)),
        out_shape=jax.ShapeDtypeStruct((8, 128), jnp.float32),
        grid=(),
    )(hist)


def kernel(x, We1, be1, We2, be2, We3, be3, codebook,
           Wd1, bd1, Wd2, bd2, Wd3, bd3):
    latent = _encoder(x, We1, be1, We2, be2, We3, be3)
    idx2d = _vq_argmin(latent, codebook)
    indices = idx2d.reshape(B)
    zeros_k = jnp.zeros((K,), jnp.float32)
    quantized, hist = _sc_gather_hist(codebook, indices, zeros_k)
    reconstruction, sse = _decoder(quantized, latent, x,
                                   Wd1, bd1, Wd2, bd2, Wd3, bd3)
    perp = _stats(hist)[0, 0]

    mean_vq = sse[0, 0] / (B * LATENT)
    vq_loss = mean_vq + CC * mean_vq
    recon_loss = sse[0, 1] / (B * INPUT_DIM)
    total_loss = vq_loss + recon_loss
    return (reconstruction, indices, vq_loss, recon_loss, total_loss, perp)


# speedup vs baseline: 1.0315x; 1.0258x over previous
import dataclasses
import functools

import jax
import jax.numpy as jnp
from jax import lax
from jax.experimental import pallas as pl
from jax.experimental.pallas import tpu as pltpu
from jax.experimental.pallas import tpu_sc as plsc

B = 4096
INPUT_DIM = 2000
HIDDEN = 1024
LATENT = 256
K = 8192
CC = 0.25

SC_CORES = 2
SC_SUBCORES = 16
SC_LANES = 16
SC_TILES = SC_CORES * SC_SUBCORES
ROWS_PER_TILE = B // SC_TILES

BM = 512
KCHUNK = 2048
NBLK = B // BM

_PARALLEL = pltpu.CompilerParams(dimension_semantics=("parallel",))


def _encvq_body(x_ref, w1_ref, b1_ref, w2_ref, b2_ref, w3_ref, b3_ref,
                cb_ref, lat_ref, idx_ref):
    h = jnp.maximum(jnp.dot(x_ref[...], w1_ref[...]) + b1_ref[...], 0.0)
    h = jnp.maximum(jnp.dot(h, w2_ref[...]) + b2_ref[...], 0.0)
    l = jnp.tanh(jnp.dot(h, w3_ref[...]) + b3_ref[...])
    lat_ref[...] = l

    a = jnp.sum(l * l, axis=1, keepdims=True)
    lm2 = l * (-2.0)

    def chunk(c, carry):
        minval, minidx = carry
        cb = cb_ref[pl.ds(c * KCHUNK, KCHUNK), :]
        bsq = jnp.sum(cb * cb, axis=1)[None, :]
        m2 = lax.dot_general(lm2, cb, (((1,), (1,)), ((), ())))
        d = (a + bsq) + m2
        cmin = jnp.min(d, axis=1, keepdims=True)
        iota = lax.broadcasted_iota(jnp.int32, d.shape, 1)
        cidx = jnp.min(jnp.where(d == cmin, iota, KCHUNK), axis=1,
                       keepdims=True) + c * KCHUNK
        upd = cmin < minval
        return (jnp.where(upd, cmin, minval), jnp.where(upd, cidx, minidx))

    init = (jnp.full((BM, 1), jnp.inf, jnp.float32),
            jnp.zeros((BM, 1), jnp.int32))
    _, minidx = lax.fori_loop(0, K // KCHUNK, chunk, init)
    idx_ref[...] = minidx


def _encvq(x, We1, be1, We2, be2, We3, be3, codebook):
    return pl.pallas_call(
        _encvq_body,
        grid=(NBLK,),
        in_specs=[
            pl.BlockSpec((BM, INPUT_DIM), lambda i: (i, 0)),
            pl.BlockSpec((INPUT_DIM, HIDDEN), lambda i: (0, 0)),
            pl.BlockSpec((1, HIDDEN), lambda i: (0, 0)),
            pl.BlockSpec((HIDDEN, HIDDEN), lambda i: (0, 0)),
            pl.BlockSpec((1, HIDDEN), lambda i: (0, 0)),
            pl.BlockSpec((HIDDEN, LATENT), lambda i: (0, 0)),
            pl.BlockSpec((1, LATENT), lambda i: (0, 0)),
            pl.BlockSpec((K, LATENT), lambda i: (0, 0)),
        ],
        out_specs=[
            pl.BlockSpec((BM, LATENT), lambda i: (i, 0)),
            pl.BlockSpec((BM, 1), lambda i: (i, 0)),
        ],
        out_shape=[
            jax.ShapeDtypeStruct((B, LATENT), jnp.float32),
            jax.ShapeDtypeStruct((B, 1), jnp.int32),
        ],
        compiler_params=_PARALLEL,
    )(x, We1, be1[None, :], We2, be2[None, :], We3, be3[None, :], codebook)


def _sc_gather_hist(codebook, indices):
    mesh = plsc.VectorSubcoreMesh(core_axis_name="c", subcore_axis_name="s")
    cp = pltpu.CompilerParams()
    if "needs_layout_passes" in pltpu.CompilerParams.__dataclass_fields__:
        cp = dataclasses.replace(cp, needs_layout_passes=False)

    @functools.partial(
        pl.kernel,
        mesh=mesh,
        compiler_params=cp,
        out_type=[
            jax.ShapeDtypeStruct((B, LATENT), jnp.float32),
            jax.ShapeDtypeStruct((1, K), jnp.float32),
        ],
        scratch_types=[
            pltpu.VMEM((ROWS_PER_TILE,), jnp.int32),
            pltpu.VMEM((ROWS_PER_TILE, LATENT), jnp.float32),
            pltpu.VMEM((B,), jnp.int32),
            pltpu.VMEM((K,), jnp.float32),
            pltpu.SemaphoreType.DMA,
        ],
    )
    def k(cb_hbm, idx_hbm, q_hbm, hist_hbm, idx_v, rows_v, idxall_v,
          cnt_v, sem):
        wid = lax.axis_index("s") * SC_CORES + lax.axis_index("c")
        base = wid * ROWS_PER_TILE
        pltpu.sync_copy(idx_hbm.at[pl.ds(base, ROWS_PER_TILE)], idx_v)
        gather = pltpu.async_copy(cb_hbm.at[idx_v], rows_v, sem)

        @pl.when(wid == 0)
        def _():
            pltpu.sync_copy(idx_hbm, idxall_v)
            zeros = jnp.zeros((SC_LANES,), jnp.float32)

            @pl.loop(0, K // SC_LANES)
            def _(i):
                cnt_v[pl.ds(i * SC_LANES, SC_LANES)] = zeros

            ones = jnp.full((SC_LANES,), 1.0, jnp.float32)

            @pl.loop(0, B // SC_LANES)
            def _(i):
                iv = idxall_v[pl.ds(i * SC_LANES, SC_LANES)]
                plsc.addupdate_scatter(cnt_v, [iv], ones)

            pltpu.sync_copy(cnt_v, hist_hbm.at[0])

        gather.wait()
        pltpu.sync_copy(rows_v, q_hbm.at[pl.ds(base, ROWS_PER_TILE)])

    return k(codebook, indices)


def _decoder_body(q_ref, lat_ref, x_ref, w1_ref, b1_ref, w2_ref, b2_ref,
                  w3_ref, b3_ref, rec_ref, sse_ref):
    q = q_ref[...]
    l = lat_ref[...]
    qst = l + (q - l)
    f32 = jnp.float32
    h = jnp.maximum(
        jnp.dot(qst.astype(jnp.bfloat16), w1_ref[...],
                preferred_element_type=f32) + b1_ref[...], 0.0)
    h = jnp.maximum(
        jnp.dot(h.astype(jnp.bfloat16), w2_ref[...],
                preferred_element_type=f32) + b2_ref[...], 0.0)
    rec = jnp.dot(h.astype(jnp.bfloat16), w3_ref[...],
                  preferred_element_type=f32) + b3_ref[...]
    rec_ref[...] = rec

    dvq = q - l
    sse_vq = jnp.sum(dvq * dvq)
    drec = rec - x_ref[...]
    sse_rec = jnp.sum(drec * drec)
    row = lax.broadcasted_iota(jnp.int32, (8, 128), 0)
    col = lax.broadcasted_iota(jnp.int32, (8, 128), 1)
    part = jnp.where((row == 0) & (col == 0), sse_vq, 0.0)
    part = part + jnp.where((row == 0) & (col == 1), sse_rec, 0.0)
    sse_ref[...] = part[None]


def _decoder(quantized, latent, x, Wd1, bd1, Wd2, bd2, Wd3, bd3):
    return pl.pallas_call(
        _decoder_body,
        grid=(NBLK,),
        in_specs=[
            pl.BlockSpec((BM, LATENT), lambda i: (i, 0)),
            pl.BlockSpec((BM, LATENT), lambda i: (i, 0)),
            pl.BlockSpec((BM, INPUT_DIM), lambda i: (i, 0)),
            pl.BlockSpec((LATENT, HIDDEN), lambda i: (0, 0)),
            pl.BlockSpec((1, HIDDEN), lambda i: (0, 0)),
            pl.BlockSpec((HIDDEN, HIDDEN), lambda i: (0, 0)),
            pl.BlockSpec((1, HIDDEN), lambda i: (0, 0)),
            pl.BlockSpec((HIDDEN, INPUT_DIM), lambda i: (0, 0)),
            pl.BlockSpec((1, INPUT_DIM), lambda i: (0, 0)),
        ],
        out_specs=[
            pl.BlockSpec((BM, INPUT_DIM), lambda i: (i, 0)),
            pl.BlockSpec((1, 8, 128), lambda i: (i, 0, 0)),
        ],
        out_shape=[
            jax.ShapeDtypeStruct((B, INPUT_DIM), jnp.float32),
            jax.ShapeDtypeStruct((NBLK, 8, 128), jnp.float32),
        ],
        compiler_params=_PARALLEL,
    )(quantized, latent, x,
      Wd1.astype(jnp.bfloat16), bd1[None, :],
      Wd2.astype(jnp.bfloat16), bd2[None, :],
      Wd3.astype(jnp.bfloat16), bd3[None, :])


def _stats_body(hist_ref, out_ref):
    avg = hist_ref[...] * (1.0 / B)
    ent = avg * jnp.log(avg + 1e-10)
    perp = jnp.exp(-jnp.sum(ent))
    out_ref[...] = jnp.full((8, 128), perp, jnp.float32)


def _stats(hist):
    return pl.pallas_call(
        _stats_body,
        in_specs=[pl.BlockSpec((1, K), lambda: (0, 0))],
        out_specs=pl.BlockSpec((8, 128), lambda: (0, 0)),
        out_shape=jax.ShapeDtypeStruct((8, 128), jnp.float32),
        grid=(),
    )(hist)


def kernel(x, We1, be1, We2, be2, We3, be3, codebook,
           Wd1, bd1, Wd2, bd2, Wd3, bd3):
    latent, idx2d = _encvq(x, We1, be1, We2, be2, We3, be3, codebook)
    indices = idx2d.reshape(B)
    quantized, hist = _sc_gather_hist(codebook, indices)
    reconstruction, sse = _decoder(quantized, latent, x,
                                   Wd1, bd1, Wd2, bd2, Wd3, bd3)
    perp = _stats(hist)[0, 0]

    mean_vq = jnp.sum(sse[:, 0, 0]) / (B * LATENT)
    vq_loss = mean_vq + CC * mean_vq
    recon_loss = jnp.sum(sse[:, 0, 1]) / (B * INPUT_DIM)
    total_loss = vq_loss + recon_loss
    return (reconstruction, indices, vq_loss, recon_loss, total_loss, perp)


# speedup vs baseline: 1.3315x; 1.2908x over previous
import dataclasses
import functools

import jax
import jax.numpy as jnp
from jax import lax
from jax.experimental import pallas as pl
from jax.experimental.pallas import tpu as pltpu
from jax.experimental.pallas import tpu_sc as plsc

B = 4096
INPUT_DIM = 2000
HIDDEN = 1024
LATENT = 256
K = 8192
CC = 0.25

SC_CORES = 2
SC_SUBCORES = 16
SC_LANES = 16
SC_TILES = SC_CORES * SC_SUBCORES
ROWS_PER_TILE = B // SC_TILES

BM = 512
KCHUNK = 2048
NBLK = B // BM


def _encvq_body(xt_ref, w1_ref, b1_ref, w2_ref, b2_ref, w3_ref, b3_ref,
                cb_ref, lat_ref, idx_ref, bsq_ref):
    @pl.when(pl.program_id(0) == 0)
    def _():
        cb = cb_ref[...]
        bsq_ref[...] = jnp.sum(cb * cb, axis=1)[None, :]

    xt = xt_ref[...]
    h = jnp.maximum(
        lax.dot_general(xt, w1_ref[...], (((0,), (0,)), ((), ())))
        + b1_ref[...], 0.0)
    h = jnp.maximum(jnp.dot(h, w2_ref[...]) + b2_ref[...], 0.0)
    l = jnp.tanh(jnp.dot(h, w3_ref[...]) + b3_ref[...])
    lat_ref[...] = l

    a = jnp.sum(l * l, axis=1, keepdims=True)
    lm2 = l * (-2.0)

    def chunk(c, carry):
        minval, minidx = carry
        cb = cb_ref[pl.ds(c * KCHUNK, KCHUNK), :]
        bsq = bsq_ref[:, pl.ds(c * KCHUNK, KCHUNK)]
        m2 = lax.dot_general(lm2, cb, (((1,), (1,)), ((), ())))
        d = (a + bsq) + m2
        cmin = jnp.min(d, axis=1, keepdims=True)
        iota = lax.broadcasted_iota(jnp.int32, d.shape, 1)
        cidx = jnp.min(jnp.where(d == cmin, iota, KCHUNK), axis=1,
                       keepdims=True) + c * KCHUNK
        upd = cmin < minval
        return (jnp.where(upd, cmin, minval), jnp.where(upd, cidx, minidx))

    init = (jnp.full((BM, 1), jnp.inf, jnp.float32),
            jnp.zeros((BM, 1), jnp.int32))
    _, minidx = lax.fori_loop(0, K // KCHUNK, chunk, init)
    idx_ref[...] = minidx


def _encvq(xt, We1, be1, We2, be2, We3, be3, codebook):
    return pl.pallas_call(
        _encvq_body,
        grid=(NBLK,),
        in_specs=[
            pl.BlockSpec((INPUT_DIM, BM), lambda i: (0, i)),
            pl.BlockSpec((INPUT_DIM, HIDDEN), lambda i: (0, 0)),
            pl.BlockSpec((1, HIDDEN), lambda i: (0, 0)),
            pl.BlockSpec((HIDDEN, HIDDEN), lambda i: (0, 0)),
            pl.BlockSpec((1, HIDDEN), lambda i: (0, 0)),
            pl.BlockSpec((HIDDEN, LATENT), lambda i: (0, 0)),
            pl.BlockSpec((1, LATENT), lambda i: (0, 0)),
            pl.BlockSpec((K, LATENT), lambda i: (0, 0)),
        ],
        out_specs=[
            pl.BlockSpec((BM, LATENT), lambda i: (i, 0)),
            pl.BlockSpec((BM, 1), lambda i: (i, 0)),
        ],
        out_shape=[
            jax.ShapeDtypeStruct((B, LATENT), jnp.float32),
            jax.ShapeDtypeStruct((B, 1), jnp.int32),
        ],
        scratch_shapes=[pltpu.VMEM((1, K), jnp.float32)],
    )(xt, We1, be1[None, :], We2, be2[None, :], We3, be3[None, :], codebook)


def _sc_gather_hist(codebook, indices):
    mesh = plsc.VectorSubcoreMesh(core_axis_name="c", subcore_axis_name="s")
    cp = pltpu.CompilerParams()
    if "needs_layout_passes" in pltpu.CompilerParams.__dataclass_fields__:
        cp = dataclasses.replace(cp, needs_layout_passes=False)

    @functools.partial(
        pl.kernel,
        mesh=mesh,
        compiler_params=cp,
        out_type=[
            jax.ShapeDtypeStruct((B, LATENT), jnp.float32),
            jax.ShapeDtypeStruct((1, K), jnp.float32),
        ],
        scratch_types=[
            pltpu.VMEM((ROWS_PER_TILE,), jnp.int32),
            pltpu.VMEM((ROWS_PER_TILE, LATENT), jnp.float32),
            pltpu.VMEM((B,), jnp.int32),
            pltpu.VMEM((K,), jnp.float32),
            pltpu.SemaphoreType.DMA,
        ],
    )
    def k(cb_hbm, idx_hbm, q_hbm, hist_hbm, idx_v, rows_v, idxall_v,
          cnt_v, sem):
        wid = lax.axis_index("s") * SC_CORES + lax.axis_index("c")
        base = wid * ROWS_PER_TILE
        pltpu.sync_copy(idx_hbm.at[pl.ds(base, ROWS_PER_TILE)], idx_v)
        gather = pltpu.async_copy(cb_hbm.at[idx_v], rows_v, sem)

        @pl.when(wid == 0)
        def _():
            pltpu.sync_copy(idx_hbm, idxall_v)
            zeros = jnp.zeros((SC_LANES,), jnp.float32)

            @pl.loop(0, K // SC_LANES)
            def _(i):
                cnt_v[pl.ds(i * SC_LANES, SC_LANES)] = zeros

            ones = jnp.full((SC_LANES,), 1.0, jnp.float32)

            @pl.loop(0, B // SC_LANES)
            def _(i):
                iv = idxall_v[pl.ds(i * SC_LANES, SC_LANES)]
                plsc.addupdate_scatter(cnt_v, [iv], ones)

            pltpu.sync_copy(cnt_v, hist_hbm.at[0])

        gather.wait()
        pltpu.sync_copy(rows_v, q_hbm.at[pl.ds(base, ROWS_PER_TILE)])

    return k(codebook, indices)


def _decoder_body(q_ref, lat_ref, xt_ref, w1_ref, b1_ref, w2_ref, b2_ref,
                  w3t_ref, b3_ref, rect_ref, sse_ref):
    q = q_ref[...]
    l = lat_ref[...]
    qst = l + (q - l)
    f32 = jnp.float32
    h = jnp.maximum(
        jnp.dot(qst.astype(jnp.bfloat16), w1_ref[...],
                preferred_element_type=f32) + b1_ref[...], 0.0)
    h = jnp.maximum(
        jnp.dot(h.astype(jnp.bfloat16), w2_ref[...],
                preferred_element_type=f32) + b2_ref[...], 0.0)
    rect = lax.dot_general(w3t_ref[...], h.astype(jnp.bfloat16),
                           (((1,), (1,)), ((), ())),
                           preferred_element_type=f32) + b3_ref[...]
    rect_ref[...] = rect

    dvq = q - l
    sse_vq = jnp.sum(dvq * dvq)
    drec = rect - xt_ref[...]
    sse_rec = jnp.sum(drec * drec)
    row = lax.broadcasted_iota(jnp.int32, (8, 128), 0)
    col = lax.broadcasted_iota(jnp.int32, (8, 128), 1)
    part = jnp.where((row == 0) & (col == 0), sse_vq, 0.0)
    part = part + jnp.where((row == 0) & (col == 1), sse_rec, 0.0)
    sse_ref[...] = part[None]


def _decoder(quantized, latent, xt, Wd1, bd1, Wd2, bd2, Wd3, bd3):
    w3t = Wd3.astype(jnp.bfloat16).T
    return pl.pallas_call(
        _decoder_body,
        grid=(NBLK,),
        in_specs=[
            pl.BlockSpec((BM, LATENT), lambda i: (i, 0)),
            pl.BlockSpec((BM, LATENT), lambda i: (i, 0)),
            pl.BlockSpec((INPUT_DIM, BM), lambda i: (0, i)),
            pl.BlockSpec((LATENT, HIDDEN), lambda i: (0, 0)),
            pl.BlockSpec((1, HIDDEN), lambda i: (0, 0)),
            pl.BlockSpec((HIDDEN, HIDDEN), lambda i: (0, 0)),
            pl.BlockSpec((1, HIDDEN), lambda i: (0, 0)),
            pl.BlockSpec((INPUT_DIM, HIDDEN), lambda i: (0, 0)),
            pl.BlockSpec((INPUT_DIM, 1), lambda i: (0, 0)),
        ],
        out_specs=[
            pl.BlockSpec((INPUT_DIM, BM), lambda i: (0, i)),
            pl.BlockSpec((1, 8, 128), lambda i: (i, 0, 0)),
        ],
        out_shape=[
            jax.ShapeDtypeStruct((INPUT_DIM, B), jnp.float32),
            jax.ShapeDtypeStruct((NBLK, 8, 128), jnp.float32),
        ],
    )(quantized, latent, xt,
      Wd1.astype(jnp.bfloat16), bd1[None, :],
      Wd2.astype(jnp.bfloat16), bd2[None, :],
      w3t, bd3[:, None])


def _stats_body(hist_ref, out_ref):
    avg = hist_ref[...] * (1.0 / B)
    ent = avg * jnp.log(avg + 1e-10)
    perp = jnp.exp(-jnp.sum(ent))
    out_ref[...] = jnp.full((8, 128), perp, jnp.float32)


def _stats(hist):
    return pl.pallas_call(
        _stats_body,
        in_specs=[pl.BlockSpec((1, K), lambda: (0, 0))],
        out_specs=pl.BlockSpec((8, 128), lambda: (0, 0)),
        out_shape=jax.ShapeDtypeStruct((8, 128), jnp.float32),
        grid=(),
    )(hist)


def kernel(x, We1, be1, We2, be2, We3, be3, codebook,
           Wd1, bd1, Wd2, bd2, Wd3, bd3):
    xt = x.T
    latent, idx2d = _encvq(xt, We1, be1, We2, be2, We3, be3, codebook)
    indices = idx2d.reshape(B)
    quantized, hist = _sc_gather_hist(codebook, indices)
    rect, sse = _decoder(quantized, latent, xt, Wd1, bd1, Wd2, bd2, Wd3, bd3)
    reconstruction = rect.T
    perp = _stats(hist)[0, 0]

    mean_vq = jnp.sum(sse[:, 0, 0]) / (B * LATENT)
    vq_loss = mean_vq + CC * mean_vq
    recon_loss = jnp.sum(sse[:, 0, 1]) / (B * INPUT_DIM)
    total_loss = vq_loss + recon_loss
    return (reconstruction, indices, vq_loss, recon_loss, total_loss, perp)
